# Initial kernel scaffold; baseline (speedup 1.0000x reference)
import jax
import jax.numpy as jnp
from jax.experimental import pallas as pl


def kernel(x, weight):
    raise NotImplementedError("write your pallas kernel here")


# speedup vs baseline: 1.7961x; 1.7961x over previous
import functools

import jax
import jax.numpy as jnp
from jax import lax
from jax.experimental import pallas as pl
from jax.experimental.pallas import tpu as pltpu
from jax.experimental.pallas import tpu_sc as plsc

_NW = 32
_B = 16384 * 32
_D = 128
_CHUNK = 128
_PER_W = _B // _NW
_NCH = _PER_W // _CHUNK

_mesh = plsc.VectorSubcoreMesh(core_axis_name="c", subcore_axis_name="s")


@functools.partial(
    pl.kernel,
    mesh=_mesh,
    out_type=jax.ShapeDtypeStruct((_B, _D), jnp.float32),
    scratch_types=[
        pltpu.VMEM((_NCH, _CHUNK), jnp.int32),
        pltpu.VMEM((_CHUNK, _D), jnp.float32),
        pltpu.SemaphoreType.DMA,
    ],
)
def _gather_all(idx_hbm, table_hbm, out_hbm, idx_v, buf, sem):
    wid = lax.axis_index("s") * 2 + lax.axis_index("c")
    base = wid * _PER_W
    pltpu.sync_copy(idx_hbm.at[wid], idx_v)

    def body(j, carry):
        pltpu.async_copy(table_hbm.at[idx_v.at[j]], buf, sem).wait()
        pltpu.sync_copy(buf, out_hbm.at[pl.ds(base + j * _CHUNK, _CHUNK)])
        return carry

    lax.fori_loop(0, _NCH, body, 0)


def kernel(x, weight):
    xr = x.reshape(_NW, _NCH, _CHUNK)
    out = _gather_all(xr, weight)
    return out.reshape(16384, 32, _D)


# speedup vs baseline: 1.8122x; 1.0089x over previous
import functools

import jax
import jax.numpy as jnp
from jax import lax
from jax.experimental import pallas as pl
from jax.experimental.pallas import tpu as pltpu
from jax.experimental.pallas import tpu_sc as plsc

_NW = 32
_B = 16384 * 32
_D = 128
_G = 128
_PER_W = _B // _NW
_NGRP = _PER_W // _G
_NT = _NGRP // 4

_mesh = plsc.VectorSubcoreMesh(core_axis_name="c", subcore_axis_name="s")


@functools.partial(
    pl.kernel,
    mesh=_mesh,
    out_type=jax.ShapeDtypeStruct((_B, _D), jnp.float32),
    scratch_types=[
        pltpu.VMEM((_NGRP, _G), jnp.int32),
        pltpu.VMEM((_G, _D), jnp.float32),
        pltpu.VMEM((_G, _D), jnp.float32),
        pltpu.VMEM((_G, _D), jnp.float32),
        pltpu.VMEM((_G, _D), jnp.float32),
        pltpu.SemaphoreType.DMA,
        pltpu.SemaphoreType.DMA,
        pltpu.SemaphoreType.DMA,
        pltpu.SemaphoreType.DMA,
        pltpu.SemaphoreType.DMA,
        pltpu.SemaphoreType.DMA,
        pltpu.SemaphoreType.DMA,
        pltpu.SemaphoreType.DMA,
    ],
)
def _gather_all(idx_hbm, table_hbm, out_hbm, idx_v,
                b0, b1, b2, b3, g0, g1, g2, g3, w0, w1, w2, w3):
    wid = lax.axis_index("s") * 2 + lax.axis_index("c")
    base = wid * _PER_W
    pltpu.sync_copy(idx_hbm.at[wid], idx_v)

    bufs = (b0, b1, b2, b3)
    gsems = (g0, g1, g2, g3)
    wsems = (w0, w1, w2, w3)

    def g_start(b, g):
        pltpu.async_copy(table_hbm.at[idx_v.at[g]], bufs[b], gsems[b])

    def g_wait(b):
        pltpu.make_async_copy(table_hbm.at[idx_v.at[0]], bufs[b],
                              gsems[b]).wait()

    def w_start(b, g):
        pltpu.async_copy(bufs[b], out_hbm.at[pl.ds(base + g * _G, _G)],
                         wsems[b])

    def w_wait(b):
        pltpu.make_async_copy(bufs[b], out_hbm.at[pl.ds(base, _G)],
                              wsems[b]).wait()

    g_start(0, 0)
    g_start(1, 1)
    for i, b in enumerate((0, 1)):
        g_wait(b)
        w_start(b, i)
    g_start(2, 2)
    g_start(3, 3)
    for i, b in enumerate((2, 3)):
        g_wait(b)
        w_start(b, 2 + i)
    for i, b in enumerate((0, 1)):
        w_wait(b)
        g_start(b, 4 + i)

    def body(t, carry):
        ga = 4 * t
        for i, b in enumerate((0, 1)):
            g_wait(b)
            w_start(b, ga + i)
        for i, b in enumerate((2, 3)):
            w_wait(b)
            g_start(b, ga + 2 + i)
        for i, b in enumerate((2, 3)):
            g_wait(b)
            w_start(b, ga + 2 + i)
        for i, b in enumerate((0, 1)):
            w_wait(b)
            g_start(b, ga + 4 + i)
        return carry

    lax.fori_loop(1, _NT - 1, body, 0)

    ga = 4 * (_NT - 1)
    for i, b in enumerate((0, 1)):
        g_wait(b)
        w_start(b, ga + i)
    for i, b in enumerate((2, 3)):
        w_wait(b)
        g_start(b, ga + 2 + i)
    for i, b in enumerate((2, 3)):
        g_wait(b)
        w_start(b, ga + 2 + i)
    for b in (0, 1, 2, 3):
        w_wait(b)


def kernel(x, weight):
    xr = x.reshape(_NW, _NGRP, _G)
    out = _gather_all(xr, weight)
    return out.reshape(16384, 32, _D)
